# baseline (speedup 1.0000x reference)
import functools
import jax
import jax.numpy as jnp
from jax import lax
from jax.experimental import pallas as pl
from jax.experimental.pallas import tpu as pltpu
from jax.experimental.pallas import tpu_sc as plsc

N = 10000
E = 160000
F = 256
G = 128

NC = 2
NS = 16
NT = NC * NS

ROW_BLK = 400

_SC_MESH = plsc.VectorSubcoreMesh(core_axis_name="c", subcore_axis_name="s",
                                  num_cores=NC, num_subcores=NS)


def _zero_vmem(ref, nwords):
    z = jnp.zeros((16,), jnp.float32)

    def body(i, _):
        ref[pl.ds(i * 16, 16)] = z
        return 0

    lax.fori_loop(0, nwords // 16, body, 0)


_DEG_CH = 125
_DEG_ROWS = 10
_DEG_NBLK = E // NC // NS // (_DEG_ROWS * _DEG_CH)


NP = 10240
_SLT = NP // NS


def _deg_sc(ei4_hbm, out_hbm, hist_sh, dstbuf, ones, zsrc):
    c = lax.axis_index("c")
    s = lax.axis_index("s")
    for i in range(8):
        ones[pl.ds(i * 16, 16)] = jnp.ones((16,), jnp.float32)
    _zero_vmem(zsrc, _SLT)
    pltpu.sync_copy(zsrc, hist_sh.at[pl.ds(s * _SLT, _SLT)])
    plsc.subcore_barrier()
    for k in range(_DEG_NBLK):
        pltpu.sync_copy(ei4_hbm.at[1].at[(c * NS + s) * _DEG_NBLK + k], dstbuf)
        for j in range(_DEG_ROWS):
            pltpu.sync_copy(ones.at[pl.ds(0, _DEG_CH)],
                            hist_sh.at[dstbuf.at[j]], add=True)
    plsc.subcore_barrier()
    pltpu.sync_copy(hist_sh.at[pl.ds(s * _SLT, _SLT)],
                    out_hbm.at[pl.ds((c * NS + s) * _SLT, _SLT)])


@functools.partial(
    pl.kernel,
    out_type=jax.ShapeDtypeStruct((NC * NP,), jnp.float32),
    mesh=_SC_MESH,
    scratch_types=[
        pltpu.VMEM_SHARED((NP,), jnp.float32),
        pltpu.VMEM((_DEG_ROWS, _DEG_CH), jnp.int32),
        pltpu.VMEM((128,), jnp.float32),
        pltpu.VMEM((_SLT,), jnp.float32),
    ],
)
def _deg_hist(ei4_hbm, out_hbm, hist_sh, dstbuf, ones, zsrc):
    _deg_sc(ei4_hbm, out_hbm, hist_sh, dstbuf, ones, zsrc)


_CV_CH = 80
_CV_ROWS = 5
_CV_BLK = _CV_CH * _CV_ROWS
_CV_NBLK = E // NS // _CV_BLK
_CV_NBUF = 4


def _conv_sc(hs2_hbm, ei4_hbm, out_hbm, acc_sh, stage, srcbuf,
             dstbuf, gsem, psem, ssem):
    c = lax.axis_index("c")
    s = lax.axis_index("s")
    def zrow(i, _):
        for j in range(8):
            stage[0, i, pl.ds(j * 16, 16)] = jnp.zeros((16,), jnp.float32)
        return 0
    lax.fori_loop(0, _CV_CH, zrow, 0)
    for k in range(_SLT // _CV_CH):
        pltpu.sync_copy(stage.at[0],
                        acc_sh.at[pl.ds(s * _SLT + k * _CV_CH, _CV_CH)])
    plsc.subcore_barrier()

    NCHUNK = _CV_NBLK * _CV_ROWS

    def _gather(t, buf):
        k, j = divmod(t, _CV_ROWS)
        return pltpu.async_copy(hs2_hbm.at[c].at[srcbuf.at[k % 2].at[j]],
                                stage.at[buf], gsem.at[buf])

    pltpu.sync_copy(ei4_hbm.at[0].at[s * _CV_NBLK], srcbuf.at[0])
    pltpu.sync_copy(ei4_hbm.at[1].at[s * _CV_NBLK], dstbuf.at[0])
    gd = {0: _gather(0, 0), 1: _gather(1, 1)}
    sc = {}
    sc_waited = -1
    pf = [None, None]
    for t in range(NCHUNK):
        k, j = divmod(t, _CV_ROWS)
        b = t % _CV_NBUF
        gd[t].wait()
        drain_to = t - 2
        if j == 0 and k + 1 < _CV_NBLK:
            drain_to = t - 1
        for u in range(sc_waited + 1, drain_to + 1):
            sc[u].wait()
        if drain_to > sc_waited:
            sc_waited = drain_to
        if j == 0 and k + 1 < _CV_NBLK:
            pf[0] = pltpu.async_copy(ei4_hbm.at[0].at[s * _CV_NBLK + k + 1],
                                     srcbuf.at[(k + 1) % 2], psem.at[0])
            pf[1] = pltpu.async_copy(ei4_hbm.at[1].at[s * _CV_NBLK + k + 1],
                                     dstbuf.at[(k + 1) % 2], psem.at[1])
        tn = t + 2
        if tn < NCHUNK:
            if tn % _CV_ROWS == 0:
                pf[0].wait()
                pf[1].wait()
            gd[tn] = _gather(tn, tn % _CV_NBUF)
        sc[t] = pltpu.async_copy(stage.at[b],
                                 acc_sh.at[dstbuf.at[k % 2].at[j]],
                                 ssem.at[b], add=True)
    sc[NCHUNK - 2].wait()
    sc[NCHUNK - 1].wait()
    plsc.subcore_barrier()
    pltpu.sync_copy(acc_sh.at[pl.ds(s * _SLT, _SLT)],
                    out_hbm.at[c].at[pl.ds(s * _SLT, _SLT)])


@functools.partial(
    pl.kernel,
    out_type=jax.ShapeDtypeStruct((NC, NP, 128), jnp.float32),
    mesh=_SC_MESH,
    scratch_types=[
        pltpu.VMEM_SHARED((NP, 128), jnp.float32),
        pltpu.VMEM((_CV_NBUF, _CV_CH, 128), jnp.float32),
        pltpu.VMEM((2, _CV_ROWS, _CV_CH), jnp.int32),
        pltpu.VMEM((2, _CV_ROWS, _CV_CH), jnp.int32),
        pltpu.SemaphoreType.DMA((_CV_NBUF,)),
        pltpu.SemaphoreType.DMA((2,)),
        pltpu.SemaphoreType.DMA((_CV_NBUF,)),
    ],
)
def _conv_agg(hs2_hbm, ei4_hbm, out_hbm, acc_sh, stage, srcbuf,
              dstbuf, gsem, psem, ssem):
    _conv_sc(hs2_hbm, ei4_hbm, out_hbm, acc_sh, stage, srcbuf,
             dstbuf, gsem, psem, ssem)


def _kA_body(x_ref, w_ref, h0_ref, h1_ref, root_ref, hs2_ref, xr_ref, s_ref):
    i = pl.program_id(0)
    dinv = jax.lax.rsqrt(h0_ref[0] + h1_ref[0] + 1.0)
    s_ref[...] = dinv
    hs = jnp.dot(x_ref[...], w_ref[...],
                 preferred_element_type=jnp.float32) * dinv
    hs2_ref[0] = hs[:, :128]
    hs2_ref[1] = hs[:, 128:]

    @pl.when(i == 0)
    def _():
        xr_ref[...] = jnp.zeros_like(xr_ref)

    ids = i * ROW_BLK + jax.lax.broadcasted_iota(jnp.int32, (1, ROW_BLK), 1)
    R = (root_ref[...] == ids).astype(jnp.float32)
    xr_ref[...] += jnp.dot(R, x_ref[...],
                           preferred_element_type=jnp.float32)


def _kernel_A(x, W1, hist3, root_col):
    return pl.pallas_call(
        _kA_body,
        grid=(N // ROW_BLK,),
        in_specs=[
            pl.BlockSpec((ROW_BLK, F), lambda i: (i, 0)),
            pl.BlockSpec((F, F), lambda i: (0, 0)),
            pl.BlockSpec((1, ROW_BLK, 1), lambda i: (0, i, 0)),
            pl.BlockSpec((1, ROW_BLK, 1), lambda i: (1, i, 0)),
            pl.BlockSpec((G, 1), lambda i: (0, 0)),
        ],
        out_specs=[
            pl.BlockSpec((NC, ROW_BLK, 128), lambda i: (0, i, 0)),
            pl.BlockSpec((G, F), lambda i: (0, 0)),
            pl.BlockSpec((ROW_BLK, 1), lambda i: (i, 0)),
        ],
        out_shape=[
            jax.ShapeDtypeStruct((NC, N, 128), jnp.float32),
            jax.ShapeDtypeStruct((G, F), jnp.float32),
            jax.ShapeDtypeStruct((N, 1), jnp.float32),
        ],
    )(x, W1, hist3, hist3, root_col)


def _rr_body(xr_ref, w_ref, o_ref):
    o_ref[...] = jnp.dot(jax.nn.relu(xr_ref[...]), w_ref[...],
                         preferred_element_type=jnp.float32)


def _kernel_rr(xr, W2b):
    return pl.pallas_call(_rr_body,
                          out_shape=jax.ShapeDtypeStruct((G, F), jnp.float32),
                          )(xr, W2b)


def _kB_body(a_ref, hs_ref, s_ref, b1_ref, w_ref, rr_ref, bat_ref, root_ref,
             zs2_ref, h1r_ref):
    i = pl.program_id(0)
    h1 = jnp.concatenate(
        [a_ref[0] + hs_ref[0], a_ref[1] + hs_ref[1]], axis=1
    ) * s_ref[...] + b1_ref[...]
    z = jnp.dot(jax.nn.relu(h1), w_ref[...], preferred_element_type=jnp.float32)
    gids = jax.lax.broadcasted_iota(jnp.int32, (1, G), 1)
    S = (bat_ref[...] == gids).astype(jnp.float32)
    z = z + jnp.dot(S, rr_ref[...], preferred_element_type=jnp.float32)
    zs = z * s_ref[...]
    zs2_ref[0] = zs[:, :128]
    zs2_ref[1] = zs[:, 128:]

    @pl.when(i == 0)
    def _():
        h1r_ref[...] = jnp.zeros_like(h1r_ref)

    ids = i * ROW_BLK + jax.lax.broadcasted_iota(jnp.int32, (1, ROW_BLK), 1)
    R = (root_ref[...] == ids).astype(jnp.float32)
    h1r_ref[...] += jnp.dot(R, h1, preferred_element_type=jnp.float32)


def _kernel_B(acc1, hs2, dinv_col, b1, W2a, rr, bat_col, root_col):
    return pl.pallas_call(
        _kB_body,
        grid=(N // ROW_BLK,),
        in_specs=[
            pl.BlockSpec((NC, ROW_BLK, 128), lambda i: (0, i, 0)),
            pl.BlockSpec((NC, ROW_BLK, 128), lambda i: (0, i, 0)),
            pl.BlockSpec((ROW_BLK, 1), lambda i: (i, 0)),
            pl.BlockSpec((1, F), lambda i: (0, 0)),
            pl.BlockSpec((F, F), lambda i: (0, 0)),
            pl.BlockSpec((G, F), lambda i: (0, 0)),
            pl.BlockSpec((ROW_BLK, 1), lambda i: (i, 0)),
            pl.BlockSpec((G, 1), lambda i: (0, 0)),
        ],
        out_specs=[
            pl.BlockSpec((NC, ROW_BLK, 128), lambda i: (0, i, 0)),
            pl.BlockSpec((G, F), lambda i: (0, 0)),
        ],
        out_shape=[
            jax.ShapeDtypeStruct((NC, N, 128), jnp.float32),
            jax.ShapeDtypeStruct((G, F), jnp.float32),
        ],
    )(acc1, hs2, dinv_col, b1, W2a, rr, bat_col, root_col)


def _kC_body(a_ref, zs_ref, s_ref, b2_ref, bat_ref, pooled_ref, cnt_ref):
    i = pl.program_id(0)
    h2 = jax.nn.relu(jnp.concatenate(
        [a_ref[0] + zs_ref[0], a_ref[1] + zs_ref[1]], axis=1
    ) * s_ref[...] + b2_ref[...])
    gids = jax.lax.broadcasted_iota(jnp.int32, (G, 1), 0)
    St = (gids == bat_ref[...].reshape(1, ROW_BLK)).astype(jnp.float32)

    @pl.when(i == 0)
    def _():
        pooled_ref[...] = jnp.zeros_like(pooled_ref)
        cnt_ref[...] = jnp.zeros_like(cnt_ref)

    pooled_ref[...] += jnp.dot(St, h2, preferred_element_type=jnp.float32)
    cnt_ref[...] += jnp.sum(St, axis=1, keepdims=True)


def _kernel_C(acc2, zs2, dinv_col, b2, bat_col):
    return pl.pallas_call(
        _kC_body,
        grid=(N // ROW_BLK,),
        in_specs=[
            pl.BlockSpec((NC, ROW_BLK, 128), lambda i: (0, i, 0)),
            pl.BlockSpec((NC, ROW_BLK, 128), lambda i: (0, i, 0)),
            pl.BlockSpec((ROW_BLK, 1), lambda i: (i, 0)),
            pl.BlockSpec((1, F), lambda i: (0, 0)),
            pl.BlockSpec((ROW_BLK, 1), lambda i: (i, 0)),
        ],
        out_specs=[
            pl.BlockSpec((G, F), lambda i: (0, 0)),
            pl.BlockSpec((G, 1), lambda i: (0, 0)),
        ],
        out_shape=[
            jax.ShapeDtypeStruct((G, F), jnp.float32),
            jax.ShapeDtypeStruct((G, 1), jnp.float32),
        ],
    )(acc2, zs2, dinv_col, b2, bat_col)


def kernel(x, edge_index, batch, root_index, W1, b1, W2, b2):
    ei4d = edge_index.reshape(2, NT * _DEG_NBLK, _DEG_ROWS, _DEG_CH)
    ei4c = edge_index.reshape(2, NS * _CV_NBLK, _CV_ROWS, _CV_CH)
    hist3 = _deg_hist(ei4d).reshape(NC, NP, 1)

    bat_col = batch[:, None]
    root_col = root_index[:, None]

    hs2, xr, dinv_col = _kernel_A(x, W1, hist3, root_col)
    acc1 = _conv_agg(hs2, ei4c)

    rr = _kernel_rr(xr, W2[F:])
    zs2, h1r = _kernel_B(acc1, hs2, dinv_col, b1[None, :], W2[:F], rr,
                         bat_col, root_col)
    acc2 = _conv_agg(zs2, ei4c)

    pooled, cnt = _kernel_C(acc2, zs2, dinv_col, b2[None, :], bat_col)
    pooled = pooled / jnp.maximum(cnt, 1.0)
    half2 = jnp.where(cnt > 0, h1r, 0.0)
    return jnp.concatenate([pooled, half2], axis=1)


# speedup vs baseline: 1.0979x; 1.0979x over previous
import functools
import jax
import jax.numpy as jnp
from jax import lax
from jax.experimental import pallas as pl
from jax.experimental.pallas import tpu as pltpu
from jax.experimental.pallas import tpu_sc as plsc

N = 10000
E = 160000
F = 256
G = 128

NC = 2
NS = 16
NT = NC * NS

ROW_BLK = 400

_SC_MESH = plsc.VectorSubcoreMesh(core_axis_name="c", subcore_axis_name="s",
                                  num_cores=NC, num_subcores=NS)


def _zero_vmem(ref, nwords):
    z = jnp.zeros((16,), jnp.float32)

    def body(i, _):
        ref[pl.ds(i * 16, 16)] = z
        return 0

    lax.fori_loop(0, nwords // 16, body, 0)


_DEG_CH = 125
_DEG_ROWS = 10
_DEG_NBLK = E // NC // NS // (_DEG_ROWS * _DEG_CH)


NP = 10240
_SLT = NP // NS


def _deg_sc(ei4_hbm, out_hbm, hist_sh, dstbuf, ones, zsrc):
    c = lax.axis_index("c")
    s = lax.axis_index("s")
    for i in range(8):
        ones[pl.ds(i * 16, 16)] = jnp.ones((16,), jnp.float32)
    _zero_vmem(zsrc, _SLT)
    pltpu.sync_copy(zsrc, hist_sh.at[pl.ds(s * _SLT, _SLT)])
    plsc.subcore_barrier()
    for k in range(_DEG_NBLK):
        pltpu.sync_copy(ei4_hbm.at[1].at[(c * NS + s) * _DEG_NBLK + k], dstbuf)
        for j in range(_DEG_ROWS):
            pltpu.sync_copy(ones.at[pl.ds(0, _DEG_CH)],
                            hist_sh.at[dstbuf.at[j]], add=True)
    plsc.subcore_barrier()
    pltpu.sync_copy(hist_sh.at[pl.ds(s * _SLT, _SLT)],
                    out_hbm.at[pl.ds((c * NS + s) * _SLT, _SLT)])


@functools.partial(
    pl.kernel,
    out_type=jax.ShapeDtypeStruct((NC * NP,), jnp.float32),
    mesh=_SC_MESH,
    scratch_types=[
        pltpu.VMEM_SHARED((NP,), jnp.float32),
        pltpu.VMEM((_DEG_ROWS, _DEG_CH), jnp.int32),
        pltpu.VMEM((128,), jnp.float32),
        pltpu.VMEM((_SLT,), jnp.float32),
    ],
)
def _deg_hist(ei4_hbm, out_hbm, hist_sh, dstbuf, ones, zsrc):
    _deg_sc(ei4_hbm, out_hbm, hist_sh, dstbuf, ones, zsrc)


_CV_CH = 100
_CV_ROWS = 10
_CV_BLK = _CV_CH * _CV_ROWS
_CV_NBLK = E // NS // _CV_BLK
_CV_NBUF = 3


def _conv_sc(hs2_hbm, ei4_hbm, out_hbm, acc_sh, stage, srcbuf,
             dstbuf, gsem, psem, ssem):
    c = lax.axis_index("c")
    s = lax.axis_index("s")
    def zrow(i, _):
        for j in range(8):
            stage[0, i, pl.ds(j * 16, 16)] = jnp.zeros((16,), jnp.float32)
        return 0
    lax.fori_loop(0, _CV_CH, zrow, 0)
    for k in range(6):
        pltpu.sync_copy(stage.at[0].at[pl.ds(0, 96)],
                        acc_sh.at[pl.ds(s * _SLT + k * 96, 96)])
    pltpu.sync_copy(stage.at[0].at[pl.ds(0, 64)],
                    acc_sh.at[pl.ds(s * _SLT + 576, 64)])
    plsc.subcore_barrier()

    NCHUNK = _CV_NBLK * _CV_ROWS

    def _gather(t, buf):
        k, j = divmod(t, _CV_ROWS)
        return pltpu.async_copy(hs2_hbm.at[c].at[srcbuf.at[k % 2].at[j]],
                                stage.at[buf], gsem.at[buf])

    pltpu.sync_copy(ei4_hbm.at[0].at[s * _CV_NBLK], srcbuf.at[0])
    pltpu.sync_copy(ei4_hbm.at[1].at[s * _CV_NBLK], dstbuf.at[0])
    gd = {0: _gather(0, 0), 1: _gather(1, 1)}
    sc = {}
    sc_waited = -1
    pf = [None, None]
    for t in range(NCHUNK):
        k, j = divmod(t, _CV_ROWS)
        b = t % _CV_NBUF
        gd[t].wait()
        drain_to = t - 1
        for u in range(sc_waited + 1, drain_to + 1):
            sc[u].wait()
        if drain_to > sc_waited:
            sc_waited = drain_to
        if j == 0 and k + 1 < _CV_NBLK:
            pf[0] = pltpu.async_copy(ei4_hbm.at[0].at[s * _CV_NBLK + k + 1],
                                     srcbuf.at[(k + 1) % 2], psem.at[0])
            pf[1] = pltpu.async_copy(ei4_hbm.at[1].at[s * _CV_NBLK + k + 1],
                                     dstbuf.at[(k + 1) % 2], psem.at[1])
        tn = t + 2
        if tn < NCHUNK:
            if tn % _CV_ROWS == 0:
                pf[0].wait()
                pf[1].wait()
            gd[tn] = _gather(tn, tn % _CV_NBUF)
        sc[t] = pltpu.async_copy(stage.at[b],
                                 acc_sh.at[dstbuf.at[k % 2].at[j]],
                                 ssem.at[b], add=True)
    sc[NCHUNK - 1].wait()
    plsc.subcore_barrier()
    pltpu.sync_copy(acc_sh.at[pl.ds(s * _SLT, _SLT)],
                    out_hbm.at[c].at[pl.ds(s * _SLT, _SLT)])


@functools.partial(
    pl.kernel,
    out_type=jax.ShapeDtypeStruct((NC, NP, 128), jnp.float32),
    mesh=_SC_MESH,
    scratch_types=[
        pltpu.VMEM_SHARED((NP, 128), jnp.float32),
        pltpu.VMEM((_CV_NBUF, _CV_CH, 128), jnp.float32),
        pltpu.VMEM((2, _CV_ROWS, _CV_CH), jnp.int32),
        pltpu.VMEM((2, _CV_ROWS, _CV_CH), jnp.int32),
        pltpu.SemaphoreType.DMA((_CV_NBUF,)),
        pltpu.SemaphoreType.DMA((2,)),
        pltpu.SemaphoreType.DMA((_CV_NBUF,)),
    ],
)
def _conv_agg(hs2_hbm, ei4_hbm, out_hbm, acc_sh, stage, srcbuf,
              dstbuf, gsem, psem, ssem):
    _conv_sc(hs2_hbm, ei4_hbm, out_hbm, acc_sh, stage, srcbuf,
             dstbuf, gsem, psem, ssem)


def _kA_body(x_ref, w_ref, h0_ref, h1_ref, root_ref, hs2_ref, xr_ref, s_ref):
    i = pl.program_id(0)
    dinv = jax.lax.rsqrt(h0_ref[0] + h1_ref[0] + 1.0)
    s_ref[...] = dinv
    hs = jnp.dot(x_ref[...], w_ref[...],
                 preferred_element_type=jnp.float32) * dinv
    hs2_ref[0] = hs[:, :128]
    hs2_ref[1] = hs[:, 128:]

    @pl.when(i == 0)
    def _():
        xr_ref[...] = jnp.zeros_like(xr_ref)

    ids = i * ROW_BLK + jax.lax.broadcasted_iota(jnp.int32, (1, ROW_BLK), 1)
    R = (root_ref[...] == ids).astype(jnp.float32)
    xr_ref[...] += jnp.dot(R, x_ref[...],
                           preferred_element_type=jnp.float32)


def _kernel_A(x, W1, hist3, root_col):
    return pl.pallas_call(
        _kA_body,
        grid=(N // ROW_BLK,),
        in_specs=[
            pl.BlockSpec((ROW_BLK, F), lambda i: (i, 0)),
            pl.BlockSpec((F, F), lambda i: (0, 0)),
            pl.BlockSpec((1, ROW_BLK, 1), lambda i: (0, i, 0)),
            pl.BlockSpec((1, ROW_BLK, 1), lambda i: (1, i, 0)),
            pl.BlockSpec((G, 1), lambda i: (0, 0)),
        ],
        out_specs=[
            pl.BlockSpec((NC, ROW_BLK, 128), lambda i: (0, i, 0)),
            pl.BlockSpec((G, F), lambda i: (0, 0)),
            pl.BlockSpec((ROW_BLK, 1), lambda i: (i, 0)),
        ],
        out_shape=[
            jax.ShapeDtypeStruct((NC, N, 128), jnp.float32),
            jax.ShapeDtypeStruct((G, F), jnp.float32),
            jax.ShapeDtypeStruct((N, 1), jnp.float32),
        ],
    )(x, W1, hist3, hist3, root_col)


def _rr_body(xr_ref, w_ref, o_ref):
    o_ref[...] = jnp.dot(jax.nn.relu(xr_ref[...]), w_ref[...],
                         preferred_element_type=jnp.float32)


def _kernel_rr(xr, W2b):
    return pl.pallas_call(_rr_body,
                          out_shape=jax.ShapeDtypeStruct((G, F), jnp.float32),
                          )(xr, W2b)


def _kB_body(a_ref, hs_ref, s_ref, b1_ref, w_ref, rr_ref, bat_ref, root_ref,
             zs2_ref, h1r_ref):
    i = pl.program_id(0)
    h1 = jnp.concatenate(
        [a_ref[0] + hs_ref[0], a_ref[1] + hs_ref[1]], axis=1
    ) * s_ref[...] + b1_ref[...]
    z = jnp.dot(jax.nn.relu(h1), w_ref[...], preferred_element_type=jnp.float32)
    gids = jax.lax.broadcasted_iota(jnp.int32, (1, G), 1)
    S = (bat_ref[...] == gids).astype(jnp.float32)
    z = z + jnp.dot(S, rr_ref[...], preferred_element_type=jnp.float32)
    zs = z * s_ref[...]
    zs2_ref[0] = zs[:, :128]
    zs2_ref[1] = zs[:, 128:]

    @pl.when(i == 0)
    def _():
        h1r_ref[...] = jnp.zeros_like(h1r_ref)

    ids = i * ROW_BLK + jax.lax.broadcasted_iota(jnp.int32, (1, ROW_BLK), 1)
    R = (root_ref[...] == ids).astype(jnp.float32)
    h1r_ref[...] += jnp.dot(R, h1, preferred_element_type=jnp.float32)


def _kernel_B(acc1, hs2, dinv_col, b1, W2a, rr, bat_col, root_col):
    return pl.pallas_call(
        _kB_body,
        grid=(N // ROW_BLK,),
        in_specs=[
            pl.BlockSpec((NC, ROW_BLK, 128), lambda i: (0, i, 0)),
            pl.BlockSpec((NC, ROW_BLK, 128), lambda i: (0, i, 0)),
            pl.BlockSpec((ROW_BLK, 1), lambda i: (i, 0)),
            pl.BlockSpec((1, F), lambda i: (0, 0)),
            pl.BlockSpec((F, F), lambda i: (0, 0)),
            pl.BlockSpec((G, F), lambda i: (0, 0)),
            pl.BlockSpec((ROW_BLK, 1), lambda i: (i, 0)),
            pl.BlockSpec((G, 1), lambda i: (0, 0)),
        ],
        out_specs=[
            pl.BlockSpec((NC, ROW_BLK, 128), lambda i: (0, i, 0)),
            pl.BlockSpec((G, F), lambda i: (0, 0)),
        ],
        out_shape=[
            jax.ShapeDtypeStruct((NC, N, 128), jnp.float32),
            jax.ShapeDtypeStruct((G, F), jnp.float32),
        ],
    )(acc1, hs2, dinv_col, b1, W2a, rr, bat_col, root_col)


def _kC_body(a_ref, zs_ref, s_ref, b2_ref, bat_ref, pooled_ref, cnt_ref):
    i = pl.program_id(0)
    h2 = jax.nn.relu(jnp.concatenate(
        [a_ref[0] + zs_ref[0], a_ref[1] + zs_ref[1]], axis=1
    ) * s_ref[...] + b2_ref[...])
    gids = jax.lax.broadcasted_iota(jnp.int32, (G, 1), 0)
    St = (gids == bat_ref[...].reshape(1, ROW_BLK)).astype(jnp.float32)

    @pl.when(i == 0)
    def _():
        pooled_ref[...] = jnp.zeros_like(pooled_ref)
        cnt_ref[...] = jnp.zeros_like(cnt_ref)

    pooled_ref[...] += jnp.dot(St, h2, preferred_element_type=jnp.float32)
    cnt_ref[...] += jnp.sum(St, axis=1, keepdims=True)


def _kernel_C(acc2, zs2, dinv_col, b2, bat_col):
    return pl.pallas_call(
        _kC_body,
        grid=(N // ROW_BLK,),
        in_specs=[
            pl.BlockSpec((NC, ROW_BLK, 128), lambda i: (0, i, 0)),
            pl.BlockSpec((NC, ROW_BLK, 128), lambda i: (0, i, 0)),
            pl.BlockSpec((ROW_BLK, 1), lambda i: (i, 0)),
            pl.BlockSpec((1, F), lambda i: (0, 0)),
            pl.BlockSpec((ROW_BLK, 1), lambda i: (i, 0)),
        ],
        out_specs=[
            pl.BlockSpec((G, F), lambda i: (0, 0)),
            pl.BlockSpec((G, 1), lambda i: (0, 0)),
        ],
        out_shape=[
            jax.ShapeDtypeStruct((G, F), jnp.float32),
            jax.ShapeDtypeStruct((G, 1), jnp.float32),
        ],
    )(acc2, zs2, dinv_col, b2, bat_col)


def kernel(x, edge_index, batch, root_index, W1, b1, W2, b2):
    ei4d = edge_index.reshape(2, NT * _DEG_NBLK, _DEG_ROWS, _DEG_CH)
    ei4c = edge_index.reshape(2, NS * _CV_NBLK, _CV_ROWS, _CV_CH)
    hist3 = _deg_hist(ei4d).reshape(NC, NP, 1)

    bat_col = batch[:, None]
    root_col = root_index[:, None]

    hs2, xr, dinv_col = _kernel_A(x, W1, hist3, root_col)
    acc1 = _conv_agg(hs2, ei4c)

    rr = _kernel_rr(xr, W2[F:])
    zs2, h1r = _kernel_B(acc1, hs2, dinv_col, b1[None, :], W2[:F], rr,
                         bat_col, root_col)
    acc2 = _conv_agg(zs2, ei4c)

    pooled, cnt = _kernel_C(acc2, zs2, dinv_col, b2[None, :], bat_col)
    pooled = pooled / jnp.maximum(cnt, 1.0)
    half2 = jnp.where(cnt > 0, h1r, 0.0)
    return jnp.concatenate([pooled, half2], axis=1)
